# Initial kernel scaffold; baseline (speedup 1.0000x reference)
import jax
import jax.numpy as jnp
from jax.experimental import pallas as pl


def kernel(t, z, sysP, eW1, eb1, eW2, eb2, nW1, nb1, nW2, nb2, gW1, gb1, gW2, gb2, qW, qb, pW, pb):
    raise NotImplementedError("write your pallas kernel here")


# speedup vs baseline: 105.3662x; 105.3662x over previous
import jax
import jax.numpy as jnp
from jax.experimental import pallas as pl
from jax.experimental.pallas import tpu as pltpu

N = 256
F = 6
K = 32
PACK = 4
NJ = N // PACK
PK = PACK * K


def _swish(a):
    return a * jax.nn.sigmoid(a)


def _ogn_kernel(x_ref, x64_ref, w1sbd_ref, w1s2_ref, w1d_ref, c1_ref,
                w2bd_ref, b2t_ref,
                nw1x_ref, nw1a_ref, nc1_ref, nw2_ref, nb2_ref,
                rw_ref, rb_ref, out_ref):
    x = x_ref[0]
    x64 = x64_ref[0]
    qm = jnp.mean(x[:, 0:2], axis=0, keepdims=True)

    w1d = w1d_ref[...]
    corrA = jnp.dot(qm, w1s2_ref[...],
                    preferred_element_type=jnp.float32)
    Aflat = (jnp.dot(x64, w1sbd_ref[...],
                     preferred_element_type=jnp.float32)
             - jnp.tile(corrA, (1, PACK)))
    B = (jnp.dot(x, w1d, preferred_element_type=jnp.float32)
         - jnp.dot(qm, w1d[0:2], preferred_element_type=jnp.float32))
    Bc = B + c1_ref[...]

    Btile = jnp.concatenate([Bc, Bc, Bc, Bc], axis=1)
    Ab = jnp.broadcast_to(Aflat[None], (N, NJ, PK)).reshape(N * NJ, PK)
    Bb = jnp.broadcast_to(Btile[:, None, :], (N, NJ, PK)).reshape(N * NJ, PK)

    h = _swish(Ab + Bb).astype(jnp.bfloat16)
    ep = _swish(jnp.dot(h, w2bd_ref[...],
                        preferred_element_type=jnp.float32) + b2t_ref[...])

    s = ep.reshape(N, NJ, PK).sum(axis=1)
    agg = s[:, 0:K] + s[:, K:2 * K] + s[:, 2 * K:3 * K] + s[:, 3 * K:4 * K]

    nw1x = nw1x_ref[...]
    nx = (jnp.dot(x, nw1x, preferred_element_type=jnp.float32)
          - jnp.dot(qm, nw1x[0:2], preferred_element_type=jnp.float32))
    h1 = _swish(nx + jnp.dot(agg, nw1a_ref[...],
                             preferred_element_type=jnp.float32) + nc1_ref[...])
    vp = _swish(jnp.dot(h1, nw2_ref[...],
                        preferred_element_type=jnp.float32) + nb2_ref[...])

    out_ref[0] = jnp.dot(vp, rw_ref[...],
                         preferred_element_type=jnp.float32) + rb_ref[...]


def kernel(t, z, sysP, eW1, eb1, eW2, eb2, nW1, nb1, nW2, nb2,
           gW1, gb1, gW2, gb2, qW, qb, pW, pb):
    bs = z.shape[0]
    n = sysP.shape[1]
    d = z.shape[1] // (2 * n)
    q = z[:, : z.shape[1] // 2].reshape(bs, n, d)
    p = z[:, z.shape[1] // 2:].reshape(bs, n, d)
    xcat = jnp.concatenate([q, p, sysP], axis=-1)
    f = xcat.shape[-1]
    x64 = xcat.reshape(bs, NJ, PACK * f)

    w1s = eW1[0:f]
    w1d = eW1[f:2 * f]
    zf = jnp.zeros((f, K), eW1.dtype)
    w1sbd = jnp.block([[w1s if i == j else zf for j in range(PACK)]
                       for i in range(PACK)])
    c1 = (eW1[2 * f] + eW1[2 * f + 1] + eb1)[None, :]
    z32 = jnp.zeros((K, K), eW2.dtype)
    w2bd = jnp.block([[eW2, z32, z32, z32],
                      [z32, eW2, z32, z32],
                      [z32, z32, eW2, z32],
                      [z32, z32, z32, eW2]]).astype(jnp.bfloat16)
    b2t = jnp.tile(eb2[None, :], (1, PACK))
    nw1x = nW1[0:f]
    nw1a = nW1[f:f + K]
    nc1 = (nW1[f + K] + nb1)[None, :]
    rw = jnp.concatenate([qW, pW], axis=1)
    rb = jnp.concatenate([qb, pb])[None, :]

    wspec2 = lambda shape: pl.BlockSpec(shape, lambda b: (0, 0))
    out = pl.pallas_call(
        _ogn_kernel,
        grid=(bs,),
        in_specs=[
            pl.BlockSpec((1, n, f), lambda b: (b, 0, 0)),
            pl.BlockSpec((1, NJ, PACK * f), lambda b: (b, 0, 0)),
            wspec2((PACK * f, PK)), wspec2((2, K)),
            wspec2((f, K)), wspec2((1, K)),
            wspec2((PK, PK)), wspec2((1, PK)),
            wspec2((f, K)), wspec2((K, K)), wspec2((1, K)),
            wspec2((K, K)), wspec2((1, K)),
            wspec2((K, 2 * d)), wspec2((1, 2 * d)),
        ],
        out_specs=pl.BlockSpec((1, n, 2 * d), lambda b: (b, 0, 0)),
        out_shape=jax.ShapeDtypeStruct((bs, n, 2 * d), jnp.float32),
        compiler_params=pltpu.CompilerParams(
            dimension_semantics=("arbitrary",)),
    )(xcat, x64, w1sbd, w1s[0:2], w1d, c1, w2bd, b2t, nw1x, nw1a, nc1,
      nW2, nb2[None, :], rw, rb)

    qdot = out[:, :, 0:d].reshape(bs, n * d)
    pdot = out[:, :, d:2 * d].reshape(bs, n * d)
    return jnp.concatenate([qdot, pdot], axis=-1)


# speedup vs baseline: 138.2420x; 1.3120x over previous
import jax
import jax.numpy as jnp
from jax.experimental import pallas as pl
from jax.experimental.pallas import tpu as pltpu

N = 256
F = 6
K = 32
PACK = 4
NJ = N // PACK
PK = PACK * K


def _swish_half(h):
    return h + h * jnp.tanh(h)


def _ogn_kernel(x_ref, x64_ref, w1sbd_ref, wq_ref, w1dt_ref, c1t_ref,
                w2bd_ref, b2t_ref,
                nw1x_ref, nw1a4_ref, nc1_ref, nw2_ref, nb2_ref,
                rw_ref, rb_ref, out_ref):
    x = x_ref[0]
    x64 = x64_ref[0]
    qm = jnp.mean(x[:, 0:2], axis=0, keepdims=True)

    corr = jnp.dot(qm, wq_ref[...],
                   preferred_element_type=jnp.float32)
    Aflat = (jnp.dot(x64, w1sbd_ref[...],
                     preferred_element_type=jnp.float32)
             - corr[:, 0:PK])
    Btile = (jnp.dot(x, w1dt_ref[...],
                     preferred_element_type=jnp.float32)
             - corr[:, PK:2 * PK] + c1t_ref[...])
    Ab = jnp.broadcast_to(Aflat[None], (N, NJ, PK)).reshape(N * NJ, PK)
    Bb = jnp.broadcast_to(Btile[:, None, :], (N, NJ, PK)).reshape(N * NJ, PK)

    h = _swish_half(Ab + Bb).astype(jnp.bfloat16)
    ep = _swish_half(jnp.dot(h, w2bd_ref[...],
                        preferred_element_type=jnp.float32) + b2t_ref[...])

    s = ep.reshape(N, NJ, PK).sum(axis=1)

    nw1x = nw1x_ref[...]
    nx = (jnp.dot(x, nw1x, preferred_element_type=jnp.float32)
          - jnp.dot(qm, nw1x[0:2], preferred_element_type=jnp.float32))
    h1 = _swish_half(nx + jnp.dot(s, nw1a4_ref[...],
                             preferred_element_type=jnp.float32) + nc1_ref[...])
    vp = _swish_half(jnp.dot(h1, nw2_ref[...],
                        preferred_element_type=jnp.float32) + nb2_ref[...])

    out_ref[0] = jnp.dot(vp, rw_ref[...],
                         preferred_element_type=jnp.float32) + rb_ref[...]


def kernel(t, z, sysP, eW1, eb1, eW2, eb2, nW1, nb1, nW2, nb2,
           gW1, gb1, gW2, gb2, qW, qb, pW, pb):
    bs = z.shape[0]
    n = sysP.shape[1]
    d = z.shape[1] // (2 * n)
    q = z[:, : z.shape[1] // 2].reshape(bs, n, d)
    p = z[:, z.shape[1] // 2:].reshape(bs, n, d)
    xcat = jnp.concatenate([q, p, sysP], axis=-1)
    f = xcat.shape[-1]
    x64 = xcat.reshape(bs, NJ, PACK * f)

    w1s = 0.5 * eW1[0:f]
    w1d = 0.5 * eW1[f:2 * f]
    zf = jnp.zeros((f, K), eW1.dtype)
    w1sbd = jnp.block([[w1s if i == j else zf for j in range(PACK)]
                       for i in range(PACK)])
    w1dt = jnp.tile(w1d, (1, PACK))
    wq = jnp.concatenate([jnp.tile(w1s[0:2], (1, PACK)),
                          jnp.tile(w1d[0:2], (1, PACK))], axis=1)
    c1 = 0.5 * (eW1[2 * f] + eW1[2 * f + 1] + eb1)[None, :]
    c1t = jnp.tile(c1, (1, PACK))
    z32 = jnp.zeros((K, K), eW2.dtype)
    w2bd = jnp.block([[eW2, z32, z32, z32],
                      [z32, eW2, z32, z32],
                      [z32, z32, eW2, z32],
                      [z32, z32, z32, eW2]])
    w2bd = (0.5 * w2bd).astype(jnp.bfloat16)
    b2t = jnp.tile(0.5 * eb2[None, :], (1, PACK))
    nw1x = 0.5 * nW1[0:f]
    nw1a4 = jnp.tile(0.5 * nW1[f:f + K], (PACK, 1))
    nc1 = 0.5 * (nW1[f + K] + nb1)[None, :]
    rw = jnp.concatenate([qW, pW], axis=1)
    rb = jnp.concatenate([qb, pb])[None, :]

    wspec2 = lambda shape: pl.BlockSpec(shape, lambda b: (0, 0))
    out = pl.pallas_call(
        _ogn_kernel,
        grid=(bs,),
        in_specs=[
            pl.BlockSpec((1, n, f), lambda b: (b, 0, 0)),
            pl.BlockSpec((1, NJ, PACK * f), lambda b: (b, 0, 0)),
            wspec2((PACK * f, PK)), wspec2((2, 2 * PK)),
            wspec2((f, PK)), wspec2((1, PK)),
            wspec2((PK, PK)), wspec2((1, PK)),
            wspec2((f, K)), wspec2((PK, K)), wspec2((1, K)),
            wspec2((K, K)), wspec2((1, K)),
            wspec2((K, 2 * d)), wspec2((1, 2 * d)),
        ],
        out_specs=pl.BlockSpec((1, n, 2 * d), lambda b: (b, 0, 0)),
        out_shape=jax.ShapeDtypeStruct((bs, n, 2 * d), jnp.float32),
        compiler_params=pltpu.CompilerParams(
            dimension_semantics=("arbitrary",)),
    )(xcat, x64, w1sbd, wq, w1dt, c1t, w2bd, b2t, nw1x, nw1a4, nc1,
      0.5 * nW2, 0.5 * nb2[None, :], rw, rb)

    qdot = out[:, :, 0:d].reshape(bs, n * d)
    pdot = out[:, :, d:2 * d].reshape(bs, n * d)
    return jnp.concatenate([qdot, pdot], axis=-1)


# speedup vs baseline: 138.4791x; 1.0017x over previous
import jax
import jax.numpy as jnp
from jax.experimental import pallas as pl
from jax.experimental.pallas import tpu as pltpu

N = 256
F = 6
K = 32
PACK = 4
NJ = N // PACK
PK = PACK * K


def _swish_half(h):
    return h + h * jnp.tanh(h)


def _ogn_kernel(x_ref, x64_ref, w1sbd_ref, wq_ref, w1dt_ref, c1t_ref,
                w2bd_ref, b2t_ref,
                nw1x_ref, nw1a4_ref, nc1_ref, nw2_ref, nb2_ref,
                rw_ref, rb_ref, out_ref):
    x = x_ref[0]
    x64 = x64_ref[0]
    qm = jnp.mean(x[:, 0:2], axis=0, keepdims=True)

    corr = jnp.dot(qm, wq_ref[...],
                   preferred_element_type=jnp.float32)
    Aflat = (jnp.dot(x64, w1sbd_ref[...],
                     preferred_element_type=jnp.float32)
             - corr[:, 0:PK])
    Btile = (jnp.dot(x, w1dt_ref[...],
                     preferred_element_type=jnp.float32)
             - corr[:, PK:2 * PK] + c1t_ref[...])
    Ab = jnp.broadcast_to(Aflat[None], (N, NJ, PK)).reshape(N * NJ, PK)
    Bb = jnp.broadcast_to(Btile[:, None, :], (N, NJ, PK)).reshape(N * NJ, PK)

    h = _swish_half(Ab + Bb).astype(jnp.bfloat16)
    ep = _swish_half(jnp.dot(h, w2bd_ref[...],
                        preferred_element_type=jnp.float32) + b2t_ref[...])

    s = ep.reshape(N, NJ, PK).sum(axis=1)

    nw1x = nw1x_ref[...]
    nx = (jnp.dot(x, nw1x, preferred_element_type=jnp.float32)
          - jnp.dot(qm, nw1x[0:2], preferred_element_type=jnp.float32))
    h1 = _swish_half(nx + jnp.dot(s, nw1a4_ref[...],
                             preferred_element_type=jnp.float32) + nc1_ref[...])
    vp = _swish_half(jnp.dot(h1, nw2_ref[...],
                        preferred_element_type=jnp.float32) + nb2_ref[...])

    out_ref[0] = jnp.dot(vp, rw_ref[...],
                         preferred_element_type=jnp.float32) + rb_ref[...]


def kernel(t, z, sysP, eW1, eb1, eW2, eb2, nW1, nb1, nW2, nb2,
           gW1, gb1, gW2, gb2, qW, qb, pW, pb):
    bs = z.shape[0]
    n = sysP.shape[1]
    d = z.shape[1] // (2 * n)
    q = z[:, : z.shape[1] // 2].reshape(bs, n, d)
    p = z[:, z.shape[1] // 2:].reshape(bs, n, d)
    xcat = jnp.concatenate([q, p, sysP], axis=-1)
    f = xcat.shape[-1]
    x64 = xcat.reshape(bs, NJ, PACK * f)

    w1s = 0.5 * eW1[0:f]
    w1d = 0.5 * eW1[f:2 * f]
    zf = jnp.zeros((f, K), eW1.dtype)
    w1sbd = jnp.block([[w1s if i == j else zf for j in range(PACK)]
                       for i in range(PACK)])
    w1dt = jnp.tile(w1d, (1, PACK))
    wq = jnp.concatenate([jnp.tile(w1s[0:2], (1, PACK)),
                          jnp.tile(w1d[0:2], (1, PACK))], axis=1)
    c1 = 0.5 * (eW1[2 * f] + eW1[2 * f + 1] + eb1)[None, :]
    c1t = jnp.tile(c1, (1, PACK))
    z32 = jnp.zeros((K, K), eW2.dtype)
    w2bd = jnp.block([[eW2, z32, z32, z32],
                      [z32, eW2, z32, z32],
                      [z32, z32, eW2, z32],
                      [z32, z32, z32, eW2]])
    w2bd = (0.5 * w2bd).astype(jnp.bfloat16)
    b2t = jnp.tile(0.5 * eb2[None, :], (1, PACK))
    nw1x = 0.5 * nW1[0:f]
    nw1a4 = jnp.tile(0.5 * nW1[f:f + K], (PACK, 1))
    nc1 = 0.5 * (nW1[f + K] + nb1)[None, :]
    rw = jnp.concatenate([qW, pW], axis=1)
    rb = jnp.concatenate([qb, pb])[None, :]

    wspec2 = lambda shape: pl.BlockSpec(shape, lambda b: (0, 0))
    out = pl.pallas_call(
        _ogn_kernel,
        grid=(bs,),
        in_specs=[
            pl.BlockSpec((1, n, f), lambda b: (b, 0, 0)),
            pl.BlockSpec((1, NJ, PACK * f), lambda b: (b, 0, 0)),
            wspec2((PACK * f, PK)), wspec2((2, 2 * PK)),
            wspec2((f, PK)), wspec2((1, PK)),
            wspec2((PK, PK)), wspec2((1, PK)),
            wspec2((f, K)), wspec2((PK, K)), wspec2((1, K)),
            wspec2((K, K)), wspec2((1, K)),
            wspec2((K, 2 * d)), wspec2((1, 2 * d)),
        ],
        out_specs=pl.BlockSpec((1, n, 2 * d), lambda b: (b, 0, 0)),
        out_shape=jax.ShapeDtypeStruct((bs, n, 2 * d), jnp.float32),
        compiler_params=pltpu.CompilerParams(
            dimension_semantics=("parallel",)),
    )(xcat, x64, w1sbd, wq, w1dt, c1t, w2bd, b2t, nw1x, nw1a4, nc1,
      0.5 * nW2, 0.5 * nb2[None, :], rw, rb)

    qdot = out[:, :, 0:d].reshape(bs, n * d)
    pdot = out[:, :, d:2 * d].reshape(bs, n * d)
    return jnp.concatenate([qdot, pdot], axis=-1)


# speedup vs baseline: 140.4752x; 1.0144x over previous
import jax
import jax.numpy as jnp
from jax.experimental import pallas as pl
from jax.experimental.pallas import tpu as pltpu

N = 256
F = 6
K = 32
PACK = 4
NJ = N // PACK
PK = PACK * K
BPB = 2


def _swish_half(h):
    return h + h * jnp.tanh(h)


def _ogn_kernel(x_ref, x64_ref, w1sbd_ref, wq_ref, w1dt_ref, c1t_ref,
                w2bd_ref, b2t_ref,
                nw1x_ref, nw1a4_ref, nc1_ref, nw2_ref, nb2_ref,
                rw_ref, rb_ref, out_ref):
  for bb in range(BPB):
    x = x_ref[bb]
    x64 = x64_ref[bb]
    qm = jnp.mean(x[:, 0:2], axis=0, keepdims=True)

    corr = jnp.dot(qm, wq_ref[...],
                   preferred_element_type=jnp.float32)
    Aflat = (jnp.dot(x64, w1sbd_ref[...],
                     preferred_element_type=jnp.float32)
             - corr[:, 0:PK])
    Btile = (jnp.dot(x, w1dt_ref[...],
                     preferred_element_type=jnp.float32)
             - corr[:, PK:2 * PK] + c1t_ref[...])
    Ab = jnp.broadcast_to(Aflat[None], (N, NJ, PK)).reshape(N * NJ, PK)
    Bb = jnp.broadcast_to(Btile[:, None, :], (N, NJ, PK)).reshape(N * NJ, PK)

    h = _swish_half(Ab + Bb).astype(jnp.bfloat16)
    ep = _swish_half(jnp.dot(h, w2bd_ref[...],
                        preferred_element_type=jnp.float32) + b2t_ref[...])

    s = ep.reshape(N, NJ, PK).sum(axis=1)

    nw1x = nw1x_ref[...]
    nx = (jnp.dot(x, nw1x, preferred_element_type=jnp.float32)
          - jnp.dot(qm, nw1x[0:2], preferred_element_type=jnp.float32))
    h1 = _swish_half(nx + jnp.dot(s, nw1a4_ref[...],
                             preferred_element_type=jnp.float32) + nc1_ref[...])
    vp = _swish_half(jnp.dot(h1, nw2_ref[...],
                        preferred_element_type=jnp.float32) + nb2_ref[...])

    out_ref[bb] = jnp.dot(vp, rw_ref[...],
                         preferred_element_type=jnp.float32) + rb_ref[...]


def kernel(t, z, sysP, eW1, eb1, eW2, eb2, nW1, nb1, nW2, nb2,
           gW1, gb1, gW2, gb2, qW, qb, pW, pb):
    bs = z.shape[0]
    n = sysP.shape[1]
    d = z.shape[1] // (2 * n)
    q = z[:, : z.shape[1] // 2].reshape(bs, n, d)
    p = z[:, z.shape[1] // 2:].reshape(bs, n, d)
    xcat = jnp.concatenate([q, p, sysP], axis=-1)
    f = xcat.shape[-1]
    x64 = xcat.reshape(bs, NJ, PACK * f)

    w1s = 0.5 * eW1[0:f]
    w1d = 0.5 * eW1[f:2 * f]
    zf = jnp.zeros((f, K), eW1.dtype)
    w1sbd = jnp.block([[w1s if i == j else zf for j in range(PACK)]
                       for i in range(PACK)])
    w1dt = jnp.tile(w1d, (1, PACK))
    wq = jnp.concatenate([jnp.tile(w1s[0:2], (1, PACK)),
                          jnp.tile(w1d[0:2], (1, PACK))], axis=1)
    c1 = 0.5 * (eW1[2 * f] + eW1[2 * f + 1] + eb1)[None, :]
    c1t = jnp.tile(c1, (1, PACK))
    z32 = jnp.zeros((K, K), eW2.dtype)
    w2bd = jnp.block([[eW2, z32, z32, z32],
                      [z32, eW2, z32, z32],
                      [z32, z32, eW2, z32],
                      [z32, z32, z32, eW2]])
    w2bd = (0.5 * w2bd).astype(jnp.bfloat16)
    b2t = jnp.tile(0.5 * eb2[None, :], (1, PACK))
    nw1x = 0.5 * nW1[0:f]
    nw1a4 = jnp.tile(0.5 * nW1[f:f + K], (PACK, 1))
    nc1 = 0.5 * (nW1[f + K] + nb1)[None, :]
    rw = jnp.concatenate([qW, pW], axis=1)
    rb = jnp.concatenate([qb, pb])[None, :]

    wspec2 = lambda shape: pl.BlockSpec(shape, lambda b: (0, 0))
    out = pl.pallas_call(
        _ogn_kernel,
        grid=(bs // BPB,),
        in_specs=[
            pl.BlockSpec((BPB, n, f), lambda b: (b, 0, 0)),
            pl.BlockSpec((BPB, NJ, PACK * f), lambda b: (b, 0, 0)),
            wspec2((PACK * f, PK)), wspec2((2, 2 * PK)),
            wspec2((f, PK)), wspec2((1, PK)),
            wspec2((PK, PK)), wspec2((1, PK)),
            wspec2((f, K)), wspec2((PK, K)), wspec2((1, K)),
            wspec2((K, K)), wspec2((1, K)),
            wspec2((K, 2 * d)), wspec2((1, 2 * d)),
        ],
        out_specs=pl.BlockSpec((BPB, n, 2 * d), lambda b: (b, 0, 0)),
        out_shape=jax.ShapeDtypeStruct((bs, n, 2 * d), jnp.float32),
        compiler_params=pltpu.CompilerParams(
            dimension_semantics=("parallel",)),
    )(xcat, x64, w1sbd, wq, w1dt, c1t, w2bd, b2t, nw1x, nw1a4, nc1,
      0.5 * nW2, 0.5 * nb2[None, :], rw, rb)

    qdot = out[:, :, 0:d].reshape(bs, n * d)
    pdot = out[:, :, d:2 * d].reshape(bs, n * d)
    return jnp.concatenate([qdot, pdot], axis=-1)


# speedup vs baseline: 141.3708x; 1.0064x over previous
import jax
import jax.numpy as jnp
from jax.experimental import pallas as pl
from jax.experimental.pallas import tpu as pltpu

N = 256
F = 6
K = 32
PACK = 4
NJ = N // PACK
PK = PACK * K
BPB = 4


def _swish_half(h):
    return h + h * jnp.tanh(h)


def _ogn_kernel(x_ref, x64_ref, w1sbd_ref, wq_ref, w1dt_ref, c1t_ref,
                w2bd_ref, b2t_ref,
                nw1x_ref, nw1a4_ref, nc1_ref, nw2_ref, nb2_ref,
                rw_ref, rb_ref, out_ref):
  for bb in range(BPB):
    x = x_ref[bb]
    x64 = x64_ref[bb]
    qm = jnp.mean(x[:, 0:2], axis=0, keepdims=True)

    corr = jnp.dot(qm, wq_ref[...],
                   preferred_element_type=jnp.float32)
    Aflat = (jnp.dot(x64, w1sbd_ref[...],
                     preferred_element_type=jnp.float32)
             - corr[:, 0:PK])
    Btile = (jnp.dot(x, w1dt_ref[...],
                     preferred_element_type=jnp.float32)
             - corr[:, PK:2 * PK] + c1t_ref[...])
    Ab = jnp.broadcast_to(Aflat[None], (N, NJ, PK)).reshape(N * NJ, PK)
    Bb = jnp.broadcast_to(Btile[:, None, :], (N, NJ, PK)).reshape(N * NJ, PK)

    h = _swish_half(Ab + Bb).astype(jnp.bfloat16)
    ep = _swish_half(jnp.dot(h, w2bd_ref[...],
                        preferred_element_type=jnp.float32) + b2t_ref[...])

    s = ep.reshape(N, NJ, PK).sum(axis=1)

    nw1x = nw1x_ref[...]
    nx = (jnp.dot(x, nw1x, preferred_element_type=jnp.float32)
          - jnp.dot(qm, nw1x[0:2], preferred_element_type=jnp.float32))
    h1 = _swish_half(nx + jnp.dot(s, nw1a4_ref[...],
                             preferred_element_type=jnp.float32) + nc1_ref[...])
    vp = _swish_half(jnp.dot(h1, nw2_ref[...],
                        preferred_element_type=jnp.float32) + nb2_ref[...])

    out_ref[bb] = jnp.dot(vp, rw_ref[...],
                         preferred_element_type=jnp.float32) + rb_ref[...]


def kernel(t, z, sysP, eW1, eb1, eW2, eb2, nW1, nb1, nW2, nb2,
           gW1, gb1, gW2, gb2, qW, qb, pW, pb):
    bs = z.shape[0]
    n = sysP.shape[1]
    d = z.shape[1] // (2 * n)
    q = z[:, : z.shape[1] // 2].reshape(bs, n, d)
    p = z[:, z.shape[1] // 2:].reshape(bs, n, d)
    xcat = jnp.concatenate([q, p, sysP], axis=-1)
    f = xcat.shape[-1]
    x64 = xcat.reshape(bs, NJ, PACK * f)

    w1s = 0.5 * eW1[0:f]
    w1d = 0.5 * eW1[f:2 * f]
    zf = jnp.zeros((f, K), eW1.dtype)
    w1sbd = jnp.block([[w1s if i == j else zf for j in range(PACK)]
                       for i in range(PACK)])
    w1dt = jnp.tile(w1d, (1, PACK))
    wq = jnp.concatenate([jnp.tile(w1s[0:2], (1, PACK)),
                          jnp.tile(w1d[0:2], (1, PACK))], axis=1)
    c1 = 0.5 * (eW1[2 * f] + eW1[2 * f + 1] + eb1)[None, :]
    c1t = jnp.tile(c1, (1, PACK))
    z32 = jnp.zeros((K, K), eW2.dtype)
    w2bd = jnp.block([[eW2, z32, z32, z32],
                      [z32, eW2, z32, z32],
                      [z32, z32, eW2, z32],
                      [z32, z32, z32, eW2]])
    w2bd = (0.5 * w2bd).astype(jnp.bfloat16)
    b2t = jnp.tile(0.5 * eb2[None, :], (1, PACK))
    nw1x = 0.5 * nW1[0:f]
    nw1a4 = jnp.tile(0.5 * nW1[f:f + K], (PACK, 1))
    nc1 = 0.5 * (nW1[f + K] + nb1)[None, :]
    rw = jnp.concatenate([qW, pW], axis=1)
    rb = jnp.concatenate([qb, pb])[None, :]

    wspec2 = lambda shape: pl.BlockSpec(shape, lambda b: (0, 0))
    out = pl.pallas_call(
        _ogn_kernel,
        grid=(bs // BPB,),
        in_specs=[
            pl.BlockSpec((BPB, n, f), lambda b: (b, 0, 0)),
            pl.BlockSpec((BPB, NJ, PACK * f), lambda b: (b, 0, 0)),
            wspec2((PACK * f, PK)), wspec2((2, 2 * PK)),
            wspec2((f, PK)), wspec2((1, PK)),
            wspec2((PK, PK)), wspec2((1, PK)),
            wspec2((f, K)), wspec2((PK, K)), wspec2((1, K)),
            wspec2((K, K)), wspec2((1, K)),
            wspec2((K, 2 * d)), wspec2((1, 2 * d)),
        ],
        out_specs=pl.BlockSpec((BPB, n, 2 * d), lambda b: (b, 0, 0)),
        out_shape=jax.ShapeDtypeStruct((bs, n, 2 * d), jnp.float32),
        compiler_params=pltpu.CompilerParams(
            dimension_semantics=("parallel",)),
    )(xcat, x64, w1sbd, wq, w1dt, c1t, w2bd, b2t, nw1x, nw1a4, nc1,
      0.5 * nW2, 0.5 * nb2[None, :], rw, rb)

    qdot = out[:, :, 0:d].reshape(bs, n * d)
    pdot = out[:, :, d:2 * d].reshape(bs, n * d)
    return jnp.concatenate([qdot, pdot], axis=-1)


# speedup vs baseline: 153.2965x; 1.0844x over previous
import jax
import jax.numpy as jnp
from jax.experimental import pallas as pl
from jax.experimental.pallas import tpu as pltpu

N = 256
K = 32
PACK = 4
NJ = N // PACK
PK = PACK * K
BPB = 4


def _swish_half(h):
    return h + h * jnp.tanh(h)


def _block_diag_mask(rows, row_blk):
    sub = jax.lax.broadcasted_iota(jnp.int32, (rows, PK), 0)
    lane = jax.lax.broadcasted_iota(jnp.int32, (rows, PK), 1)
    return (sub // row_blk) == (lane // K)


def _ogn_kernel(z4_ref, z64_ref, sp_ref, sp64_ref,
                ew1_ref, eb1_ref, ew2_ref, eb2_ref,
                nw1_ref, nb1_ref, nw2_ref, nb2_ref,
                qw_ref, qb_ref, pw_ref, pb_ref, out_ref):
    w1 = 0.5 * ew1_ref[...]
    w1s, w1d = w1[0:6], w1[6:12]
    c1 = w1[12:13] + w1[13:14] + 0.5 * eb1_ref[...]
    c1t = jnp.tile(c1, (1, PACK))
    w1d_t = jnp.tile(w1d, (1, PACK))
    w1sq_t = jnp.tile(w1s[0:2], (1, PACK))
    w1dq_t = jnp.tile(w1d[0:2], (1, PACK))
    m8 = _block_diag_mask(2 * PACK, 2)
    q_bd = jnp.where(m8, jnp.tile(w1s[0:2], (PACK, PACK)), 0.0)
    p_bd = jnp.where(m8, jnp.tile(w1s[2:4], (PACK, PACK)), 0.0)
    s_bd = jnp.where(m8, jnp.tile(w1s[4:6], (PACK, PACK)), 0.0)
    w2bd = jnp.where(_block_diag_mask(PK, K),
                     jnp.tile(0.5 * ew2_ref[...], (PACK, PACK)),
                     0.0).astype(jnp.bfloat16)
    b2t = jnp.tile(0.5 * eb2_ref[...], (1, PACK))
    n1 = 0.5 * nw1_ref[...]
    nw1x = n1[0:6]
    nw1a4 = jnp.tile(n1[6:6 + K], (PACK, 1))
    nc1 = n1[6 + K:7 + K] + 0.5 * nb1_ref[...]
    nw2 = 0.5 * nw2_ref[...]
    nb2 = 0.5 * nb2_ref[...]

    for bb in range(BPB):
        q = z4_ref[bb, 0]
        p = z4_ref[bb, 1]
        sp = sp_ref[bb]
        x = jnp.concatenate([q, p, sp], axis=1)
        qm = jnp.mean(q, axis=0, keepdims=True)

        Aflat = (jnp.dot(z64_ref[bb, 0], q_bd, preferred_element_type=jnp.float32)
                 + jnp.dot(z64_ref[bb, 1], p_bd, preferred_element_type=jnp.float32)
                 + jnp.dot(sp64_ref[bb], s_bd, preferred_element_type=jnp.float32)
                 - jnp.dot(qm, w1sq_t, preferred_element_type=jnp.float32))
        Btile = (jnp.dot(x, w1d_t, preferred_element_type=jnp.float32)
                 - jnp.dot(qm, w1dq_t, preferred_element_type=jnp.float32)
                 + c1t)

        Ab = jnp.broadcast_to(Aflat[None], (N, NJ, PK)).reshape(N * NJ, PK)
        Bb = jnp.broadcast_to(Btile[:, None, :], (N, NJ, PK)).reshape(N * NJ, PK)

        h = _swish_half(Ab + Bb).astype(jnp.bfloat16)
        ep = _swish_half(jnp.dot(h, w2bd,
                                 preferred_element_type=jnp.float32) + b2t)

        s = ep.reshape(N, NJ, PK).sum(axis=1)

        nx = (jnp.dot(x, nw1x, preferred_element_type=jnp.float32)
              - jnp.dot(qm, nw1x[0:2], preferred_element_type=jnp.float32))
        h1 = _swish_half(nx + jnp.dot(s, nw1a4,
                                      preferred_element_type=jnp.float32) + nc1)
        vp = _swish_half(jnp.dot(h1, nw2,
                                 preferred_element_type=jnp.float32) + nb2)

        out_ref[bb, 0] = jnp.dot(vp, qw_ref[...],
                                 preferred_element_type=jnp.float32) + qb_ref[...]
        out_ref[bb, 1] = jnp.dot(vp, pw_ref[...],
                                 preferred_element_type=jnp.float32) + pb_ref[...]


def kernel(t, z, sysP, eW1, eb1, eW2, eb2, nW1, nb1, nW2, nb2,
           gW1, gb1, gW2, gb2, qW, qb, pW, pb):
    bs = z.shape[0]
    n = sysP.shape[1]
    d = z.shape[1] // (2 * n)
    sd = sysP.shape[2]
    z4 = z.reshape(bs, 2, n, d)
    z64 = z.reshape(bs, 2, NJ, PACK * d)
    sp64 = sysP.reshape(bs, NJ, PACK * sd)

    wspec = lambda *shape: pl.BlockSpec(shape, lambda b: (0,) * len(shape))
    out = pl.pallas_call(
        _ogn_kernel,
        grid=(bs // BPB,),
        in_specs=[
            pl.BlockSpec((BPB, 2, n, d), lambda b: (b, 0, 0, 0)),
            pl.BlockSpec((BPB, 2, NJ, PACK * d), lambda b: (b, 0, 0, 0)),
            pl.BlockSpec((BPB, n, sd), lambda b: (b, 0, 0)),
            pl.BlockSpec((BPB, NJ, PACK * sd), lambda b: (b, 0, 0)),
            wspec(14, K), wspec(1, K),
            wspec(K, K), wspec(1, K),
            wspec(39, K), wspec(1, K),
            wspec(K, K), wspec(1, K),
            wspec(K, d), wspec(1, d),
            wspec(K, d), wspec(1, d),
        ],
        out_specs=pl.BlockSpec((BPB, 2, n, d), lambda b: (b, 0, 0, 0)),
        out_shape=jax.ShapeDtypeStruct((bs, 2, n, d), jnp.float32),
        compiler_params=pltpu.CompilerParams(
            dimension_semantics=("parallel",)),
    )(z4, z64, sysP, sp64,
      eW1, eb1.reshape(1, K), eW2, eb2.reshape(1, K),
      nW1, nb1.reshape(1, K), nW2, nb2.reshape(1, K),
      qW, qb.reshape(1, d), pW, pb.reshape(1, d))

    return out.reshape(bs, 2 * n * d)


# speedup vs baseline: 166.4993x; 1.0861x over previous
import jax
import jax.numpy as jnp
from jax.experimental import pallas as pl
from jax.experimental.pallas import tpu as pltpu

N = 256
K = 32
PACK = 4
NJ = N // PACK
PK = PACK * K
BPB = 4


def _swish_half(h):
    return h + h * jnp.tanh(h)


def _block_diag_mask(rows, row_blk):
    sub = jax.lax.broadcasted_iota(jnp.int32, (rows, PK), 0)
    lane = jax.lax.broadcasted_iota(jnp.int32, (rows, PK), 1)
    return (sub // row_blk) == (lane // K)


def _ogn_kernel(z4_ref, z64_ref, sp_ref, sp64_ref,
                ew1_ref, eb1_ref, ew2_ref, eb2_ref,
                nw1_ref, nb1_ref, nw2_ref, nb2_ref,
                qw_ref, qb_ref, pw_ref, pb_ref, out_ref):
    w1 = 0.5 * ew1_ref[...]
    w1s, w1d = w1[0:6], w1[6:12]
    c1 = w1[12:13] + w1[13:14] + 0.5 * eb1_ref[...]
    c1t = jnp.tile(c1, (1, PACK))
    w1d_t = jnp.tile(w1d, (1, PACK))
    w1sq_t = jnp.tile(w1s[0:2], (1, PACK))
    w1dq_t = jnp.tile(w1d[0:2], (1, PACK))
    m8 = _block_diag_mask(2 * PACK, 2)
    q_bd = jnp.where(m8, jnp.tile(w1s[0:2], (PACK, PACK)), 0.0)
    p_bd = jnp.where(m8, jnp.tile(w1s[2:4], (PACK, PACK)), 0.0)
    s_bd = jnp.where(m8, jnp.tile(w1s[4:6], (PACK, PACK)), 0.0)
    w2bd = jnp.where(_block_diag_mask(PK, K),
                     jnp.tile(0.5 * ew2_ref[...], (PACK, PACK)),
                     0.0).astype(jnp.bfloat16)
    b2t = jnp.tile(0.5 * eb2_ref[...], (1, PACK))
    n1 = 0.5 * nw1_ref[...]
    nw1x = n1[0:6]
    nw1a4 = jnp.tile(n1[6:6 + K], (PACK, 1))
    nc1 = n1[6 + K:7 + K] + 0.5 * nb1_ref[...]
    nw2 = 0.5 * nw2_ref[...]
    nb2 = 0.5 * nb2_ref[...]

    for bb in range(BPB):
        q = z4_ref[bb, 0]
        p = z4_ref[bb, 1]
        sp = sp_ref[bb]
        x = jnp.concatenate([q, p, sp], axis=1)
        qm = jnp.mean(q, axis=0, keepdims=True)

        Aflat = (jnp.dot(z64_ref[bb, 0], q_bd, preferred_element_type=jnp.float32)
                 + jnp.dot(z64_ref[bb, 1], p_bd, preferred_element_type=jnp.float32)
                 + jnp.dot(sp64_ref[bb], s_bd, preferred_element_type=jnp.float32)
                 - jnp.dot(qm, w1sq_t, preferred_element_type=jnp.float32))
        Btile = (jnp.dot(x, w1d_t, preferred_element_type=jnp.float32)
                 - jnp.dot(qm, w1dq_t, preferred_element_type=jnp.float32)
                 + c1t)

        Afb = Aflat.astype(jnp.bfloat16)
        Btb = Btile.astype(jnp.bfloat16)
        Ab = jnp.broadcast_to(Afb[None], (N, NJ, PK)).reshape(N * NJ, PK)
        Bb = jnp.broadcast_to(Btb[:, None, :], (N, NJ, PK)).reshape(N * NJ, PK)

        preb = Ab + Bb
        h = preb + preb * jnp.tanh(preb)
        ep = _swish_half(jnp.dot(h, w2bd,
                                 preferred_element_type=jnp.float32) + b2t)

        s = ep.reshape(N, NJ, PK).sum(axis=1)

        nx = (jnp.dot(x, nw1x, preferred_element_type=jnp.float32)
              - jnp.dot(qm, nw1x[0:2], preferred_element_type=jnp.float32))
        h1 = _swish_half(nx + jnp.dot(s, nw1a4,
                                      preferred_element_type=jnp.float32) + nc1)
        vp = _swish_half(jnp.dot(h1, nw2,
                                 preferred_element_type=jnp.float32) + nb2)

        out_ref[bb, 0] = jnp.dot(vp, qw_ref[...],
                                 preferred_element_type=jnp.float32) + qb_ref[...]
        out_ref[bb, 1] = jnp.dot(vp, pw_ref[...],
                                 preferred_element_type=jnp.float32) + pb_ref[...]


def kernel(t, z, sysP, eW1, eb1, eW2, eb2, nW1, nb1, nW2, nb2,
           gW1, gb1, gW2, gb2, qW, qb, pW, pb):
    bs = z.shape[0]
    n = sysP.shape[1]
    d = z.shape[1] // (2 * n)
    sd = sysP.shape[2]
    z4 = z.reshape(bs, 2, n, d)
    z64 = z.reshape(bs, 2, NJ, PACK * d)
    sp64 = sysP.reshape(bs, NJ, PACK * sd)

    wspec = lambda *shape: pl.BlockSpec(shape, lambda b: (0,) * len(shape))
    out = pl.pallas_call(
        _ogn_kernel,
        grid=(bs // BPB,),
        in_specs=[
            pl.BlockSpec((BPB, 2, n, d), lambda b: (b, 0, 0, 0)),
            pl.BlockSpec((BPB, 2, NJ, PACK * d), lambda b: (b, 0, 0, 0)),
            pl.BlockSpec((BPB, n, sd), lambda b: (b, 0, 0)),
            pl.BlockSpec((BPB, NJ, PACK * sd), lambda b: (b, 0, 0)),
            wspec(14, K), wspec(1, K),
            wspec(K, K), wspec(1, K),
            wspec(39, K), wspec(1, K),
            wspec(K, K), wspec(1, K),
            wspec(K, d), wspec(1, d),
            wspec(K, d), wspec(1, d),
        ],
        out_specs=pl.BlockSpec((BPB, 2, n, d), lambda b: (b, 0, 0, 0)),
        out_shape=jax.ShapeDtypeStruct((bs, 2, n, d), jnp.float32),
        compiler_params=pltpu.CompilerParams(
            dimension_semantics=("parallel",)),
    )(z4, z64, sysP, sp64,
      eW1, eb1.reshape(1, K), eW2, eb2.reshape(1, K),
      nW1, nb1.reshape(1, K), nW2, nb2.reshape(1, K),
      qW, qb.reshape(1, d), pW, pb.reshape(1, d))

    return out.reshape(bs, 2 * n * d)
